# baseline (speedup 1.0000x reference)
import jax
import jax.numpy as jnp
from jax import lax
from jax.experimental import pallas as pl
from jax.experimental.pallas import tpu as pltpu
from jax.experimental.pallas import tpu_sc as plsc

VOCAB = 100000
EMBD = 128
B = 4096
L = 50
LPAD = 64

NC = 2
NS = 16
NW = NC * NS

SEQ_W = B // NW
NR = 8
LA = 4


def _embed_kernel(x_hbm, table_hbm, out_hbm, idx_v, rows_v, gsem, ssem):
    wid = lax.axis_index("s") * NC + lax.axis_index("c")
    seq0 = wid * SEQ_W
    pltpu.sync_copy(x_hbm.at[pl.ds(seq0, SEQ_W)], idx_v)

    def start_gather(g, b):
        pltpu.async_copy(
            table_hbm.at[idx_v.at[g, pl.ds(0, L)]], rows_v.at[b], gsem.at[b]
        )

    def wait_gather(g, b):
        pltpu.make_async_copy(
            table_hbm.at[idx_v.at[g, pl.ds(0, L)]], rows_v.at[b], gsem.at[b]
        ).wait()

    def _store_desc(g, b):
        return pltpu.make_async_copy(
            rows_v.at[b], out_hbm.at[seq0 + g], ssem.at[b]
        )

    def start_store(g, b):
        _store_desc(g, b).start()

    def wait_store(g, b):
        _store_desc(g, b).wait()

    for b in range(LA):
        start_gather(b, b)

    for b in range(NR):
        g = b
        wait_gather(g, b)
        start_store(g, b)
        h, hb = g + LA, (b + LA) % NR
        if g >= LA:
            wait_store(h - NR, hb)
        start_gather(h, hb)

    @pl.loop(NR, SEQ_W - NR, step=NR)
    def _pass(g0):
        for b in range(NR):
            g = g0 + b
            wait_gather(g, b)
            start_store(g, b)
            h, hb = g + LA, (b + LA) % NR
            wait_store(h - NR, hb)
            start_gather(h, hb)

    for b in range(NR):
        g = SEQ_W - NR + b
        wait_gather(g, b)
        start_store(g, b)
        h, hb = g + LA, (b + LA) % NR
        if h < SEQ_W:
            wait_store(h - NR, hb)
            start_gather(h, hb)

    for b in range(NR):
        wait_store(SEQ_W - NR + b, b)


@jax.jit
def _embed(x, table):
    xp = jnp.pad(x, ((0, 0), (0, LPAD - L)))
    mesh = plsc.VectorSubcoreMesh(
        core_axis_name="c", subcore_axis_name="s", num_cores=NC,
        num_subcores=NS,
    )
    return pl.kernel(
        _embed_kernel,
        out_type=jax.ShapeDtypeStruct((B, L, EMBD), jnp.float32),
        mesh=mesh,
        scratch_types=[
            pltpu.VMEM((SEQ_W, LPAD), jnp.int32),
            pltpu.VMEM((NR, L, EMBD), jnp.float32),
            pltpu.SemaphoreType.DMA((NR,)),
            pltpu.SemaphoreType.DMA((NR,)),
        ],
        compiler_params=pltpu.CompilerParams(use_tc_tiling_on_sc=True),
    )(xp, table)


def kernel(x, table):
    return _embed(x.astype(jnp.int32), table)


# speedup vs baseline: 1.7573x; 1.7573x over previous
import jax
import jax.numpy as jnp
from jax import lax
from jax.experimental import pallas as pl
from jax.experimental.pallas import tpu as pltpu
from jax.experimental.pallas import tpu_sc as plsc

VOCAB = 100000
EMBD = 128
B = 4096
L = 50

NC = 2
NS = 16
NW = NC * NS

BW = B // NW
NR = 5
LA = 2


def _embed_kernel(xt_hbm, table_hbm, out_hbm, idx_v, rows_v, gsem, ssem):
    wid = lax.axis_index("s") * NC + lax.axis_index("c")
    b0 = wid * BW
    pltpu.sync_copy(xt_hbm.at[:, pl.ds(b0, BW)], idx_v)

    def start_gather(g, b):
        pltpu.async_copy(table_hbm.at[idx_v.at[g]], rows_v.at[b], gsem.at[b])

    def wait_gather(g, b):
        pltpu.make_async_copy(
            table_hbm.at[idx_v.at[g]], rows_v.at[b], gsem.at[b]
        ).wait()

    def _store_desc(g, b):
        return pltpu.make_async_copy(
            rows_v.at[b], out_hbm.at[g, pl.ds(b0, BW)], ssem.at[b]
        )

    def start_store(g, b):
        _store_desc(g, b).start()

    def wait_store(g, b):
        _store_desc(g, b).wait()

    for b in range(LA):
        start_gather(b, b)

    for b in range(NR):
        g = b
        wait_gather(g, b)
        start_store(g, b)
        h, hb = g + LA, (b + LA) % NR
        if h >= NR:
            wait_store(h - NR, hb)
        start_gather(h, hb)

    @pl.loop(NR, L - NR, step=NR)
    def _pass(g0):
        for b in range(NR):
            g = g0 + b
            wait_gather(g, b)
            start_store(g, b)
            h, hb = g + LA, (b + LA) % NR
            wait_store(h - NR, hb)
            start_gather(h, hb)

    for b in range(NR):
        g = L - NR + b
        wait_gather(g, b)
        start_store(g, b)
        h, hb = g + LA, (b + LA) % NR
        if h < L:
            wait_store(h - NR, hb)
            start_gather(h, hb)

    for b in range(NR):
        wait_store(L - NR + b, b)


@jax.jit
def _embed(x, table):
    xt = jnp.swapaxes(x, 0, 1)
    mesh = plsc.VectorSubcoreMesh(
        core_axis_name="c", subcore_axis_name="s", num_cores=NC,
        num_subcores=NS,
    )
    out = pl.kernel(
        _embed_kernel,
        out_type=jax.ShapeDtypeStruct((L, B, EMBD), jnp.float32),
        mesh=mesh,
        scratch_types=[
            pltpu.VMEM((L, BW), jnp.int32),
            pltpu.VMEM((NR, BW, EMBD), jnp.float32),
            pltpu.SemaphoreType.DMA((NR,)),
            pltpu.SemaphoreType.DMA((NR,)),
        ],
    )(xt, table)
    return jnp.swapaxes(out, 0, 1)


def kernel(x, table):
    return _embed(x.astype(jnp.int32), table)


# speedup vs baseline: 1.7968x; 1.0225x over previous
import jax
import jax.numpy as jnp
from jax import lax
from jax.experimental import pallas as pl
from jax.experimental.pallas import tpu as pltpu
from jax.experimental.pallas import tpu_sc as plsc

VOCAB = 100000
EMBD = 128
B = 4096
L = 50

NC = 2
NS = 16
NW = NC * NS

BW = B // NW
NR = 5
LA = 3


def _embed_kernel(xt_hbm, table_hbm, out_hbm, idx_v, rows_v, gsem, ssem):
    wid = lax.axis_index("s") * NC + lax.axis_index("c")
    b0 = wid * BW
    pltpu.sync_copy(xt_hbm.at[:, pl.ds(b0, BW)], idx_v)

    def start_gather(g, b):
        pltpu.async_copy(table_hbm.at[idx_v.at[g]], rows_v.at[b], gsem.at[b])

    def wait_gather(g, b):
        pltpu.make_async_copy(
            table_hbm.at[idx_v.at[g]], rows_v.at[b], gsem.at[b]
        ).wait()

    def _store_desc(g, b):
        return pltpu.make_async_copy(
            rows_v.at[b], out_hbm.at[g, pl.ds(b0, BW)], ssem.at[b]
        )

    def start_store(g, b):
        _store_desc(g, b).start()

    def wait_store(g, b):
        _store_desc(g, b).wait()

    for b in range(LA):
        start_gather(b, b)

    for b in range(NR):
        g = b
        wait_gather(g, b)
        start_store(g, b)
        h, hb = g + LA, (b + LA) % NR
        if h >= NR:
            wait_store(h - NR, hb)
        start_gather(h, hb)

    @pl.loop(NR, L - NR, step=NR)
    def _pass(g0):
        for b in range(NR):
            g = g0 + b
            wait_gather(g, b)
            start_store(g, b)
            h, hb = g + LA, (b + LA) % NR
            wait_store(h - NR, hb)
            start_gather(h, hb)

    for b in range(NR):
        g = L - NR + b
        wait_gather(g, b)
        start_store(g, b)
        h, hb = g + LA, (b + LA) % NR
        if h < L:
            wait_store(h - NR, hb)
            start_gather(h, hb)

    for b in range(NR):
        wait_store(L - NR + b, b)


@jax.jit
def _embed(x, table):
    xt = jnp.swapaxes(x, 0, 1)
    mesh = plsc.VectorSubcoreMesh(
        core_axis_name="c", subcore_axis_name="s", num_cores=NC,
        num_subcores=NS,
    )
    out = pl.kernel(
        _embed_kernel,
        out_type=jax.ShapeDtypeStruct((L, B, EMBD), jnp.float32),
        mesh=mesh,
        scratch_types=[
            pltpu.VMEM((L, BW), jnp.int32),
            pltpu.VMEM((NR, BW, EMBD), jnp.float32),
            pltpu.SemaphoreType.DMA((NR,)),
            pltpu.SemaphoreType.DMA((NR,)),
        ],
    )(xt, table)
    return jnp.swapaxes(out, 0, 1)


def kernel(x, table):
    return _embed(x.astype(jnp.int32), table)
